# baseline (speedup 1.0000x reference)
import functools

import jax
import jax.numpy as jnp
from jax import lax
from jax.experimental import pallas as pl
from jax.experimental.pallas import tpu as pltpu
from jax.experimental.pallas import tpu_sc as plsc

N = 10000
E = 320000
D = 128
DH = D // 2
R = 50
RP = 64
NC = 2
NS = 16
NW = NC * NS
EPW = E // NW
KC = 80
NCHUNKC = EPW // KC
EPT = E // NS
K = 80
NCHUNK = EPT // K
RPT = 624
CPT = (N * RP) // NS
_TAIL = N - NS * RPT
_F32 = jnp.float32

_MESH = plsc.VectorSubcoreMesh(core_axis_name="c", subcore_axis_name="s")
_SC_PARAMS = pltpu.CompilerParams(use_tc_tiling_on_sc=False)


@functools.partial(
    pl.kernel,
    out_type=[jax.ShapeDtypeStruct((N * RP,), _F32)] * NC,
    mesh=_MESH,
    compiler_params=_SC_PARAMS,
    scratch_types=[
        pltpu.VMEM((NCHUNKC, KC), jnp.int32),
        pltpu.VMEM((NCHUNKC, KC), jnp.int32),
        pltpu.VMEM((KC,), _F32),
        pltpu.VMEM((8000,), _F32),
        pltpu.VMEM_SHARED((N * RP,), _F32),
    ],
)
def _sc_count(dst_hbm, typ_hbm, out0_hbm, out1_hbm, dst_v, fidx_v, ones_v,
              zbuf, acc):
    cid = lax.axis_index("c")
    sid = lax.axis_index("s")
    wid = cid * NS + sid
    pltpu.sync_copy(dst_hbm.at[wid], dst_v)
    pltpu.sync_copy(typ_hbm.at[wid], fidx_v)

    def fuse(j, _):
        for s in range(KC // 16):
            sl = pl.ds(s * 16, 16)
            fidx_v[j, sl] = dst_v[j, sl] * RP + fidx_v[j, sl]
        return 0

    lax.fori_loop(0, NCHUNKC, fuse, 0)

    one16 = jnp.ones((16,), _F32)
    for s in range(KC // 16):
        ones_v[pl.ds(s * 16, 16)] = one16

    zero16 = jnp.zeros((16,), _F32)

    def zb(i, _):
        zbuf[pl.ds(i * 16, 16)] = zero16
        return 0

    lax.fori_loop(0, 8000 // 16, zb, 0)
    for b in range(CPT // 8000):
        pltpu.sync_copy(zbuf, acc.at[pl.ds(sid * CPT + b * 8000, 8000)])
    plsc.subcore_barrier()

    def ch(j, _):
        pltpu.sync_copy(ones_v, acc.at[fidx_v.at[j]], add=True)
        return 0

    lax.fori_loop(0, NCHUNKC, ch, 0)
    plsc.subcore_barrier()

    @pl.when(jnp.logical_and(sid == 0, cid == 0))
    def _():
        pltpu.sync_copy(acc, out0_hbm)

    @pl.when(jnp.logical_and(sid == 0, cid == 1))
    def _():
        pltpu.sync_copy(acc, out1_hbm)


_GB = 4


def _segsum_scratch():
    return [
        pltpu.VMEM((NCHUNK, K), jnp.int32),
        pltpu.VMEM((NCHUNK, K), jnp.int32),
    ] + [pltpu.VMEM((K, DH), _F32)] * (2 * _GB) + [
        pltpu.VMEM((16, DH), _F32),
        pltpu.VMEM_SHARED((N, DH), _F32),
        pltpu.SemaphoreType.DMA,
        pltpu.SemaphoreType.DMA,
        pltpu.SemaphoreType.DMA,
        pltpu.SemaphoreType.DMA,
    ]


def _segsum_prologue(src_hbm, dst_hbm, src_v, dst_v, zbuf, acc):
    cid = lax.axis_index("c")
    sid = lax.axis_index("s")
    pltpu.sync_copy(src_hbm.at[sid], src_v)
    pltpu.sync_copy(dst_hbm.at[sid], dst_v)

    bias = cid * N

    def add_bias(j, _):
        for s in range(K // 16):
            sl = pl.ds(s * 16, 16)
            src_v[j, sl] = src_v[j, sl] + bias
        return 0

    lax.fori_loop(0, NCHUNK, add_bias, 0)

    zero16 = jnp.zeros((16,), _F32)

    def zrow(r, _):
        for s in range(DH // 16):
            zbuf[r, pl.ds(s * 16, 16)] = zero16
        return 0

    lax.fori_loop(0, 16, zrow, 0)
    for b in range(RPT // 16):
        pltpu.sync_copy(zbuf, acc.at[pl.ds(sid * RPT + b * 16, 16)])

    @pl.when(sid == NS - 1)
    def _():
        pltpu.sync_copy(zbuf.at[pl.ds(0, _TAIL)], acc.at[pl.ds(NS * RPT, _TAIL)])

    plsc.subcore_barrier()
    return cid, sid


def _segsum_epilogue(out0_hbm, out1_hbm, acc, cid, sid):
    plsc.subcore_barrier()
    for c, out in ((0, out0_hbm), (1, out1_hbm)):
        @pl.when(cid == c)
        def _(out=out):
            pltpu.sync_copy(acc.at[pl.ds(sid * RPT, RPT)],
                            out.at[pl.ds(sid * RPT, RPT)])

            @pl.when(sid == NS - 1)
            def _():
                pltpu.sync_copy(acc.at[pl.ds(NS * RPT, _TAIL)],
                                out.at[pl.ds(NS * RPT, _TAIL)])


_NGROUP = NCHUNK // _GB
_NGITER = (_NGROUP - 2) // 2
_NLEFT = NCHUNK - _NGROUP * _GB


def _segsum_pipeline(x_hbm, src_v, bufs, acc, ga, gb, sa, sb,
                     work_chunk, dst_idx):
    A, B = bufs[:_GB], bufs[_GB:]

    def fire_gathers(base, grp, sem):
        for b in range(_GB):
            pltpu.async_copy(x_hbm.at[src_v.at[base + b]], grp[b], sem)

    def drain_gathers(base, grp, sem):
        for b in range(_GB):
            pltpu.make_async_copy(x_hbm.at[src_v.at[base + b]], grp[b],
                                  sem).wait()

    def fire_scatters(base, grp, so, sem):
        for b in range(_GB):
            pltpu.async_copy(grp[b], acc.at[dst_idx(base + b, so + b)], sem,
                             add=True)

    def drain_scatters(base, grp, so, sem):
        for b in range(_GB):
            pltpu.make_async_copy(grp[b], acc.at[dst_idx(base + b, so + b)],
                                  sem).wait()

    def process(base, grp, so, gsem, ssem):
        drain_gathers(base, grp, gsem)
        for b in range(_GB):
            work_chunk(grp[b], base + b, so + b)
        fire_scatters(base, grp, so, ssem)

    fire_gathers(0, A, ga)
    fire_gathers(_GB, B, gb)

    def it(i, _):
        g0 = i * 2 * _GB
        g1 = g0 + _GB
        process(g0, A, 0, ga, sa)
        process(g1, B, _GB, gb, sb)
        drain_scatters(g0, A, 0, sa)
        fire_gathers(g0 + 2 * _GB, A, ga)
        drain_scatters(g1, B, _GB, sb)
        fire_gathers(g1 + 2 * _GB, B, gb)
        return 0

    lax.fori_loop(0, _NGITER, it, 0)
    base_a = _NGITER * 2 * _GB
    base_b = base_a + _GB
    process(base_a, A, 0, ga, sa)
    process(base_b, B, _GB, gb, sb)
    drain_scatters(base_a, A, 0, sa)
    left = _NGROUP * _GB
    for b in range(_NLEFT):
        pltpu.async_copy(x_hbm.at[src_v.at[left + b]], A[b], ga)
    drain_scatters(base_b, B, _GB, sb)
    for b in range(_NLEFT):
        pltpu.make_async_copy(x_hbm.at[src_v.at[left + b]], A[b], ga).wait()
        work_chunk(A[b], left + b, b)
        pltpu.async_copy(A[b], acc.at[dst_idx(left + b, b)], sa, add=True)
    for b in range(_NLEFT):
        pltpu.make_async_copy(A[b], acc.at[dst_idx(left + b, b)], sa).wait()


@functools.partial(
    pl.kernel,
    out_type=[jax.ShapeDtypeStruct((N, DH), _F32)] * NC,
    mesh=_MESH,
    compiler_params=_SC_PARAMS,
    scratch_types=_segsum_scratch(),
)
def _sc_segsum(x_hbm, src_hbm, dst_hbm, out0_hbm, out1_hbm,
               src_v, dst_v, b0, b1, b2, b3, b4, b5, b6, b7,
               zbuf, acc, ga, gb, sa, sb):
    cid, sid = _segsum_prologue(src_hbm, dst_hbm, src_v, dst_v, zbuf, acc)
    _segsum_pipeline(x_hbm, src_v, (b0, b1, b2, b3, b4, b5, b6, b7),
                     acc, ga, gb, sa, sb,
                     lambda gbuf, j, slot: None,
                     lambda j, slot: dst_v.at[j])
    _segsum_epilogue(out0_hbm, out1_hbm, acc, cid, sid)


MGB = 2
_MNGROUP = 2 * (NCHUNK // (2 * MGB))
_MNGITER = (_MNGROUP - 2) // 2
_MNLEFT = NCHUNK - _MNGROUP * MGB


@functools.partial(
    pl.kernel,
    out_type=[jax.ShapeDtypeStruct((N, DH), _F32)] * NC,
    mesh=_MESH,
    compiler_params=_SC_PARAMS,
    scratch_types=[
        pltpu.VMEM((NCHUNK, K), jnp.int32),
        pltpu.VMEM((NCHUNK, K), jnp.int32),
    ] + [pltpu.VMEM((K, DH), _F32)] * (2 * MGB)
      + [pltpu.VMEM((K, DH), _F32)] * (2 * MGB) + [
        pltpu.VMEM((16, DH), _F32),
        pltpu.VMEM_SHARED((N, DH), _F32),
        pltpu.SemaphoreType.DMA,
        pltpu.SemaphoreType.DMA,
        pltpu.SemaphoreType.DMA,
        pltpu.SemaphoreType.DMA,
        pltpu.VMEM((2 * MGB, K), jnp.int32),
        pltpu.VMEM((2 * MGB, K), jnp.int32),
    ],
)
def _sc_segsum_mul(x_hbm, src_hbm, comb_hbm, relf_hbm, out0_hbm, out1_hbm,
                   src_v, comb_v, g0b, g1b, g2b, g3b, r0b, r1b, r2b, r3b,
                   zbuf, acc, ga, gb, sa, sb, dstidx_v, typidx_v):
    cid = lax.axis_index("c")
    sid = lax.axis_index("s")
    pltpu.sync_copy(src_hbm.at[sid], src_v)
    pltpu.sync_copy(comb_hbm.at[sid], comb_v)

    bias = cid * N

    def add_bias(j, _):
        for s in range(K // 16):
            sl = pl.ds(s * 16, 16)
            src_v[j, sl] = src_v[j, sl] + bias
        return 0

    lax.fori_loop(0, NCHUNK, add_bias, 0)

    zero16 = jnp.zeros((16,), _F32)

    def zrow(r, _):
        for s in range(DH // 16):
            zbuf[r, pl.ds(s * 16, 16)] = zero16
        return 0

    lax.fori_loop(0, 16, zrow, 0)
    for b in range(RPT // 16):
        pltpu.sync_copy(zbuf, acc.at[pl.ds(sid * RPT + b * 16, 16)])

    @pl.when(sid == NS - 1)
    def _():
        pltpu.sync_copy(zbuf.at[pl.ds(0, _TAIL)], acc.at[pl.ds(NS * RPT, _TAIL)])

    plsc.subcore_barrier()

    gbufs = (g0b, g1b, g2b, g3b)
    rbufs = (r0b, r1b, r2b, r3b)
    trel = cid * R

    def fire_group(base, so, gsem):
        for b in range(MGB):
            for g in range(K // 16):
                gsl = pl.ds(g * 16, 16)
                typidx_v[so + b, gsl] = (comb_v[base + b, gsl] & 63) + trel
        for b in range(MGB):
            pltpu.async_copy(x_hbm.at[src_v.at[base + b]], gbufs[so + b], gsem)
        for b in range(MGB):
            pltpu.async_copy(relf_hbm.at[typidx_v.at[so + b]], rbufs[so + b],
                             gsem)

    def drain_group(base, so, gsem):
        for b in range(MGB):
            pltpu.make_async_copy(x_hbm.at[src_v.at[base + b]], gbufs[so + b],
                                  gsem).wait()
        for b in range(MGB):
            pltpu.make_async_copy(relf_hbm.at[typidx_v.at[so + b]],
                                  rbufs[so + b], gsem).wait()

    def mul_chunk(gbuf, rbuf):
        @plsc.parallel_loop(0, K, unroll=4)
        def _(rr):
            xs = [gbuf[rr, pl.ds(s * 16, 16)] for s in range(DH // 16)]
            rl = [rbuf[rr, pl.ds(s * 16, 16)] for s in range(DH // 16)]
            for s in range(DH // 16):
                gbuf[rr, pl.ds(s * 16, 16)] = xs[s] * rl[s]

    def process(base, so, gsem, ssem):
        drain_group(base, so, gsem)
        for b in range(MGB):
            for g in range(K // 16):
                gsl = pl.ds(g * 16, 16)
                dstidx_v[so + b, gsl] = jnp.right_shift(comb_v[base + b, gsl], 6)
            pass
        for b in range(MGB):
            pltpu.async_copy(gbufs[so + b], acc.at[dstidx_v.at[so + b]], ssem,
                             add=True)

    def drain_scatters(so, ssem):
        for b in range(MGB):
            pltpu.make_async_copy(gbufs[so + b], acc.at[dstidx_v.at[so + b]],
                                  ssem).wait()

    fire_group(0, 0, ga)
    fire_group(MGB, MGB, gb)

    def it(i, _):
        g0 = i * 2 * MGB
        g1 = g0 + MGB
        process(g0, 0, ga, sa)
        process(g1, MGB, gb, sb)
        drain_scatters(0, sa)
        fire_group(g0 + 2 * MGB, 0, ga)
        drain_scatters(MGB, sb)
        fire_group(g1 + 2 * MGB, MGB, gb)
        return 0

    lax.fori_loop(0, _MNGITER, it, 0)
    base_a = _MNGITER * 2 * MGB
    process(base_a, 0, ga, sa)
    process(base_a + MGB, MGB, gb, sb)
    drain_scatters(0, sa)
    drain_scatters(MGB, sb)
    left = _MNGROUP * MGB
    for b in range(_MNLEFT):
        for g in range(K // 16):
            gsl = pl.ds(g * 16, 16)
            typidx_v[b, gsl] = (comb_v[left + b, gsl] & 63) + trel
            dstidx_v[b, gsl] = jnp.right_shift(comb_v[left + b, gsl], 6)
        pltpu.sync_copy(x_hbm.at[src_v.at[left + b]], gbufs[b])
        pltpu.sync_copy(relf_hbm.at[typidx_v.at[b]], rbufs[b])
        mul_chunk(gbufs[b], rbufs[b])
        pltpu.sync_copy(gbufs[b], acc.at[dstidx_v.at[b]], add=True)
    _segsum_epilogue(out0_hbm, out1_hbm, acc, cid, sid)


BM = 2000


def _tc_T_body(c0_ref, c1_ref, rp_ref, o_ref):
    o_ref[...] = jnp.dot(c0_ref[...] + c1_ref[...], rp_ref[...],
                         preferred_element_type=_F32)


def _tc_T(c0, c1, relpad):
    return pl.pallas_call(
        _tc_T_body,
        grid=(N // BM,),
        in_specs=[
            pl.BlockSpec((BM, RP), lambda i: (i, 0)),
            pl.BlockSpec((BM, RP), lambda i: (i, 0)),
            pl.BlockSpec((RP, D), lambda i: (0, 0)),
        ],
        out_specs=pl.BlockSpec((BM, D), lambda i: (i, 0)),
        out_shape=jax.ShapeDtypeStruct((N, D), _F32),
    )(c0, c1, relpad)


def _tc_step_body(p_ref, t_ref, a0_ref, a1_ref, x_ref, wm_ref, ws_ref, e_ref,
                  xo_ref, eo_ref):
    sign = p_ref[0, 0]
    w = p_ref[0, 1]
    agg = jnp.concatenate([a0_ref[...], a1_ref[...]], axis=1)
    agg = agg + sign * t_ref[...]
    h = jnp.dot(agg, wm_ref[...], preferred_element_type=_F32)
    h = h + jnp.dot(x_ref[...], ws_ref[...], preferred_element_type=_F32)
    xn = jnp.maximum(h, 0.0)
    xo_ref[...] = xn
    eo_ref[...] = e_ref[...] + w * xn


def _tc_step(params, t, a0, a1, x, wm, ws, ent_in):
    return pl.pallas_call(
        _tc_step_body,
        grid=(N // BM,),
        in_specs=[
            pl.BlockSpec((1, 2), lambda i: (0, 0)),
            pl.BlockSpec((BM, D), lambda i: (i, 0)),
            pl.BlockSpec((BM, DH), lambda i: (i, 0)),
            pl.BlockSpec((BM, DH), lambda i: (i, 0)),
            pl.BlockSpec((BM, D), lambda i: (i, 0)),
            pl.BlockSpec((D, D), lambda i: (0, 0)),
            pl.BlockSpec((D, D), lambda i: (0, 0)),
            pl.BlockSpec((BM, D), lambda i: (i, 0)),
        ],
        out_specs=[pl.BlockSpec((BM, D), lambda i: (i, 0))] * 2,
        out_shape=[jax.ShapeDtypeStruct((N, D), _F32)] * 2,
    )(params, t, a0, a1, x, wm, ws, ent_in)


def _tc_rel_body(w_ref, rel_ref, wr_ref, o_ref):
    wc = (w_ref[0, 0] * wr_ref[0] + w_ref[0, 1] * wr_ref[1]
          + w_ref[0, 2] * wr_ref[2])
    o_ref[...] = jnp.dot(rel_ref[...], wc, preferred_element_type=_F32)


def _tc_rel(w2d, rel, wr):
    return pl.pallas_call(
        _tc_rel_body,
        in_specs=[
            pl.BlockSpec((1, 3), lambda: (0, 0)),
            pl.BlockSpec((R, D), lambda: (0, 0)),
            pl.BlockSpec((3, D, D), lambda: (0, 0, 0)),
        ],
        out_specs=pl.BlockSpec((R, D), lambda: (0, 0)),
        out_shape=jax.ShapeDtypeStruct((R, D), _F32),
    )(w2d, rel, wr)


def _split(x):
    return jnp.concatenate([x[:, :DH], x[:, DH:]], axis=0)


def kernel(x, edge_index, edge_type, rel_embeds, weights, W_msg, W_self, W_rel):
    src_c = edge_index[0].reshape(NW, NCHUNKC, KC)
    dst_c = edge_index[1].reshape(NW, NCHUNKC, KC)
    typ_c = edge_type.reshape(NW, NCHUNKC, KC)
    src = edge_index[0].reshape(NS, NCHUNK, K)
    dst = edge_index[1].reshape(NS, NCHUNK, K)
    comb = dst * RP + edge_type.reshape(NS, NCHUNK, K)
    relf = _split(rel_embeds)
    relpad = jnp.zeros((RP, D), _F32).at[:R].set(rel_embeds)

    c0, c1 = _sc_count(dst_c, typ_c)
    t_mat = _tc_T(c0.reshape(N, RP), c1.reshape(N, RP), relpad)

    signs = jnp.array([-1.0, 0.0, 1.0], _F32)
    params = jnp.stack([signs, weights.astype(_F32)], axis=1)

    ent = jnp.zeros((N, D), _F32)
    a0, a1 = _sc_segsum(_split(x), src, dst)
    x1, ent = _tc_step(params[0].reshape(1, 2), t_mat, a0, a1, x,
                       W_msg[0], W_self[0], ent)
    a0, a1 = _sc_segsum_mul(_split(x1), src, comb, relf)
    x2, ent = _tc_step(params[1].reshape(1, 2), t_mat, a0, a1, x1,
                       W_msg[1], W_self[1], ent)
    a0, a1 = _sc_segsum(_split(x2), src, dst)
    _, ent = _tc_step(params[2].reshape(1, 2), t_mat, a0, a1, x2,
                      W_msg[2], W_self[2], ent)

    rel_out = _tc_rel(weights.astype(_F32).reshape(1, 3), rel_embeds, W_rel)
    return (ent, rel_out)


# speedup vs baseline: 1.3340x; 1.3340x over previous
import functools

import jax
import jax.numpy as jnp
from jax import lax
from jax.experimental import pallas as pl
from jax.experimental.pallas import tpu as pltpu
from jax.experimental.pallas import tpu_sc as plsc

N = 10000
E = 320000
D = 128
DH = D // 2
R = 50
RP = 64
NC = 2
NS = 16
NW = NC * NS
EPW = E // NW
KC = 80
NCHUNKC = EPW // KC
EPT = E // NS
K = 80
NCHUNK = EPT // K
RPT = 624
CPT = (N * RP) // NS
_TAIL = N - NS * RPT
_F32 = jnp.float32

_MESH = plsc.VectorSubcoreMesh(core_axis_name="c", subcore_axis_name="s")
_SC_PARAMS = pltpu.CompilerParams(use_tc_tiling_on_sc=False)


@functools.partial(
    pl.kernel,
    out_type=[jax.ShapeDtypeStruct((N * RP,), _F32)] * NC,
    mesh=_MESH,
    compiler_params=_SC_PARAMS,
    scratch_types=[
        pltpu.VMEM((NCHUNKC, KC), jnp.int32),
        pltpu.VMEM((NCHUNKC, KC), jnp.int32),
        pltpu.VMEM((KC,), _F32),
        pltpu.VMEM((8000,), _F32),
        pltpu.VMEM_SHARED((N * RP,), _F32),
    ],
)
def _sc_count(dst_hbm, typ_hbm, out0_hbm, out1_hbm, dst_v, fidx_v, ones_v,
              zbuf, acc):
    cid = lax.axis_index("c")
    sid = lax.axis_index("s")
    wid = cid * NS + sid
    pltpu.sync_copy(dst_hbm.at[wid], dst_v)
    pltpu.sync_copy(typ_hbm.at[wid], fidx_v)

    def fuse(j, _):
        for s in range(KC // 16):
            sl = pl.ds(s * 16, 16)
            fidx_v[j, sl] = dst_v[j, sl] * RP + fidx_v[j, sl]
        return 0

    lax.fori_loop(0, NCHUNKC, fuse, 0)

    one16 = jnp.ones((16,), _F32)
    for s in range(KC // 16):
        ones_v[pl.ds(s * 16, 16)] = one16

    zero16 = jnp.zeros((16,), _F32)

    def zb(i, _):
        zbuf[pl.ds(i * 16, 16)] = zero16
        return 0

    lax.fori_loop(0, 8000 // 16, zb, 0)
    for b in range(CPT // 8000):
        pltpu.sync_copy(zbuf, acc.at[pl.ds(sid * CPT + b * 8000, 8000)])
    plsc.subcore_barrier()

    def ch(j, _):
        pltpu.sync_copy(ones_v, acc.at[fidx_v.at[j]], add=True)
        return 0

    lax.fori_loop(0, NCHUNKC, ch, 0)
    plsc.subcore_barrier()

    @pl.when(jnp.logical_and(sid == 0, cid == 0))
    def _():
        pltpu.sync_copy(acc, out0_hbm)

    @pl.when(jnp.logical_and(sid == 0, cid == 1))
    def _():
        pltpu.sync_copy(acc, out1_hbm)


_GB = 4


def _segsum_scratch():
    return [
        pltpu.VMEM((NCHUNK, K), jnp.int32),
        pltpu.VMEM((NCHUNK, K), jnp.int32),
    ] + [pltpu.VMEM((K, DH), _F32)] * (2 * _GB) + [
        pltpu.VMEM((16, DH), _F32),
        pltpu.VMEM_SHARED((N, DH), _F32),
        pltpu.SemaphoreType.DMA,
        pltpu.SemaphoreType.DMA,
        pltpu.SemaphoreType.DMA,
        pltpu.SemaphoreType.DMA,
    ]


def _segsum_prologue(src_hbm, dst_hbm, src_v, dst_v, zbuf, acc):
    cid = lax.axis_index("c")
    sid = lax.axis_index("s")
    pltpu.sync_copy(src_hbm.at[sid], src_v)
    pltpu.sync_copy(dst_hbm.at[sid], dst_v)

    bias = cid * N

    def add_bias(j, _):
        for s in range(K // 16):
            sl = pl.ds(s * 16, 16)
            src_v[j, sl] = src_v[j, sl] + bias
        return 0

    lax.fori_loop(0, NCHUNK, add_bias, 0)

    zero16 = jnp.zeros((16,), _F32)

    def zrow(r, _):
        for s in range(DH // 16):
            zbuf[r, pl.ds(s * 16, 16)] = zero16
        return 0

    lax.fori_loop(0, 16, zrow, 0)
    for b in range(RPT // 16):
        pltpu.sync_copy(zbuf, acc.at[pl.ds(sid * RPT + b * 16, 16)])

    @pl.when(sid == NS - 1)
    def _():
        pltpu.sync_copy(zbuf.at[pl.ds(0, _TAIL)], acc.at[pl.ds(NS * RPT, _TAIL)])

    plsc.subcore_barrier()
    return cid, sid


def _segsum_epilogue(out0_hbm, out1_hbm, acc, cid, sid):
    plsc.subcore_barrier()
    for c, out in ((0, out0_hbm), (1, out1_hbm)):
        @pl.when(cid == c)
        def _(out=out):
            pltpu.sync_copy(acc.at[pl.ds(sid * RPT, RPT)],
                            out.at[pl.ds(sid * RPT, RPT)])

            @pl.when(sid == NS - 1)
            def _():
                pltpu.sync_copy(acc.at[pl.ds(NS * RPT, _TAIL)],
                                out.at[pl.ds(NS * RPT, _TAIL)])


_NGROUP = NCHUNK // _GB
_NGITER = (_NGROUP - 2) // 2
_NLEFT = NCHUNK - _NGROUP * _GB


def _segsum_pipeline(x_hbm, src_v, bufs, acc, ga, gb, sa, sb,
                     work_chunk, dst_idx):
    A, B = bufs[:_GB], bufs[_GB:]

    def fire_gathers(base, grp, sem):
        for b in range(_GB):
            pltpu.async_copy(x_hbm.at[src_v.at[base + b]], grp[b], sem)

    def drain_gathers(base, grp, sem):
        for b in range(_GB):
            pltpu.make_async_copy(x_hbm.at[src_v.at[base + b]], grp[b],
                                  sem).wait()

    def fire_scatters(base, grp, so, sem):
        for b in range(_GB):
            pltpu.async_copy(grp[b], acc.at[dst_idx(base + b, so + b)], sem,
                             add=True)

    def drain_scatters(base, grp, so, sem):
        for b in range(_GB):
            pltpu.make_async_copy(grp[b], acc.at[dst_idx(base + b, so + b)],
                                  sem).wait()

    def process(base, grp, so, gsem, ssem):
        drain_gathers(base, grp, gsem)
        for b in range(_GB):
            work_chunk(grp[b], base + b, so + b)
        fire_scatters(base, grp, so, ssem)

    fire_gathers(0, A, ga)
    fire_gathers(_GB, B, gb)

    def it(i, _):
        g0 = i * 2 * _GB
        g1 = g0 + _GB
        process(g0, A, 0, ga, sa)
        process(g1, B, _GB, gb, sb)
        drain_scatters(g0, A, 0, sa)
        fire_gathers(g0 + 2 * _GB, A, ga)
        drain_scatters(g1, B, _GB, sb)
        fire_gathers(g1 + 2 * _GB, B, gb)
        return 0

    lax.fori_loop(0, _NGITER, it, 0)
    base_a = _NGITER * 2 * _GB
    base_b = base_a + _GB
    process(base_a, A, 0, ga, sa)
    process(base_b, B, _GB, gb, sb)
    drain_scatters(base_a, A, 0, sa)
    left = _NGROUP * _GB
    for b in range(_NLEFT):
        pltpu.async_copy(x_hbm.at[src_v.at[left + b]], A[b], ga)
    drain_scatters(base_b, B, _GB, sb)
    for b in range(_NLEFT):
        pltpu.make_async_copy(x_hbm.at[src_v.at[left + b]], A[b], ga).wait()
        work_chunk(A[b], left + b, b)
        pltpu.async_copy(A[b], acc.at[dst_idx(left + b, b)], sa, add=True)
    for b in range(_NLEFT):
        pltpu.make_async_copy(A[b], acc.at[dst_idx(left + b, b)], sa).wait()


@functools.partial(
    pl.kernel,
    out_type=[jax.ShapeDtypeStruct((N, DH), _F32)] * NC,
    mesh=_MESH,
    compiler_params=_SC_PARAMS,
    scratch_types=_segsum_scratch(),
)
def _sc_segsum(x_hbm, src_hbm, dst_hbm, out0_hbm, out1_hbm,
               src_v, dst_v, b0, b1, b2, b3, b4, b5, b6, b7,
               zbuf, acc, ga, gb, sa, sb):
    cid, sid = _segsum_prologue(src_hbm, dst_hbm, src_v, dst_v, zbuf, acc)
    _segsum_pipeline(x_hbm, src_v, (b0, b1, b2, b3, b4, b5, b6, b7),
                     acc, ga, gb, sa, sb,
                     lambda gbuf, j, slot: None,
                     lambda j, slot: dst_v.at[j])
    _segsum_epilogue(out0_hbm, out1_hbm, acc, cid, sid)


MGB = 2
_MNGROUP = 2 * (NCHUNK // (2 * MGB))
_MNGITER = (_MNGROUP - 2) // 2
_MNLEFT = NCHUNK - _MNGROUP * MGB


@functools.partial(
    pl.kernel,
    out_type=[jax.ShapeDtypeStruct((N, DH), _F32)] * NC,
    mesh=_MESH,
    compiler_params=_SC_PARAMS,
    scratch_types=[
        pltpu.VMEM((NCHUNK, K), jnp.int32),
        pltpu.VMEM((NCHUNK, K), jnp.int32),
    ] + [pltpu.VMEM((K, DH), _F32)] * (2 * MGB)
      + [pltpu.VMEM((K, DH), _F32)] * (2 * MGB) + [
        pltpu.VMEM((16, DH), _F32),
        pltpu.VMEM_SHARED((N, DH), _F32),
        pltpu.SemaphoreType.DMA,
        pltpu.SemaphoreType.DMA,
        pltpu.SemaphoreType.DMA,
        pltpu.SemaphoreType.DMA,
        pltpu.VMEM((2 * MGB, K), jnp.int32),
        pltpu.VMEM((2 * MGB, K), jnp.int32),
    ],
)
def _sc_segsum_mul(x_hbm, src_hbm, comb_hbm, relf_hbm, out0_hbm, out1_hbm,
                   src_v, comb_v, g0b, g1b, g2b, g3b, r0b, r1b, r2b, r3b,
                   zbuf, acc, ga, gb, sa, sb, dstidx_v, typidx_v):
    cid = lax.axis_index("c")
    sid = lax.axis_index("s")
    pltpu.sync_copy(src_hbm.at[sid], src_v)
    pltpu.sync_copy(comb_hbm.at[sid], comb_v)

    bias = cid * N

    def add_bias(j, _):
        for s in range(K // 16):
            sl = pl.ds(s * 16, 16)
            src_v[j, sl] = src_v[j, sl] + bias
        return 0

    lax.fori_loop(0, NCHUNK, add_bias, 0)

    zero16 = jnp.zeros((16,), _F32)

    def zrow(r, _):
        for s in range(DH // 16):
            zbuf[r, pl.ds(s * 16, 16)] = zero16
        return 0

    lax.fori_loop(0, 16, zrow, 0)
    for b in range(RPT // 16):
        pltpu.sync_copy(zbuf, acc.at[pl.ds(sid * RPT + b * 16, 16)])

    @pl.when(sid == NS - 1)
    def _():
        pltpu.sync_copy(zbuf.at[pl.ds(0, _TAIL)], acc.at[pl.ds(NS * RPT, _TAIL)])

    plsc.subcore_barrier()

    gbufs = (g0b, g1b, g2b, g3b)
    rbufs = (r0b, r1b, r2b, r3b)
    trel = (cid * NS + sid) * R

    def fire_group(base, so, gsem):
        for b in range(MGB):
            for g in range(K // 16):
                gsl = pl.ds(g * 16, 16)
                typidx_v[so + b, gsl] = (comb_v[base + b, gsl] & 63) + trel
        for b in range(MGB):
            pltpu.async_copy(x_hbm.at[src_v.at[base + b]], gbufs[so + b], gsem)
        for b in range(MGB):
            pltpu.async_copy(relf_hbm.at[typidx_v.at[so + b]], rbufs[so + b],
                             gsem)

    def drain_group(base, so, gsem):
        for b in range(MGB):
            pltpu.make_async_copy(x_hbm.at[src_v.at[base + b]], gbufs[so + b],
                                  gsem).wait()
        for b in range(MGB):
            pltpu.make_async_copy(relf_hbm.at[typidx_v.at[so + b]],
                                  rbufs[so + b], gsem).wait()

    def mul_chunk(gbuf, rbuf):
        @plsc.parallel_loop(0, K, unroll=4)
        def _(rr):
            xs = [gbuf[rr, pl.ds(s * 16, 16)] for s in range(DH // 16)]
            rl = [rbuf[rr, pl.ds(s * 16, 16)] for s in range(DH // 16)]
            for s in range(DH // 16):
                gbuf[rr, pl.ds(s * 16, 16)] = xs[s] * rl[s]

    def process(base, so, gsem, ssem):
        drain_group(base, so, gsem)
        for b in range(MGB):
            for g in range(K // 16):
                gsl = pl.ds(g * 16, 16)
                dstidx_v[so + b, gsl] = jnp.right_shift(comb_v[base + b, gsl], 6)
            mul_chunk(gbufs[so + b], rbufs[so + b])
        for b in range(MGB):
            pltpu.async_copy(gbufs[so + b], acc.at[dstidx_v.at[so + b]], ssem,
                             add=True)

    def drain_scatters(so, ssem):
        for b in range(MGB):
            pltpu.make_async_copy(gbufs[so + b], acc.at[dstidx_v.at[so + b]],
                                  ssem).wait()

    fire_group(0, 0, ga)
    fire_group(MGB, MGB, gb)

    def it(i, _):
        g0 = i * 2 * MGB
        g1 = g0 + MGB
        process(g0, 0, ga, sa)
        process(g1, MGB, gb, sb)
        drain_scatters(0, sa)
        fire_group(g0 + 2 * MGB, 0, ga)
        drain_scatters(MGB, sb)
        fire_group(g1 + 2 * MGB, MGB, gb)
        return 0

    lax.fori_loop(0, _MNGITER, it, 0)
    base_a = _MNGITER * 2 * MGB
    process(base_a, 0, ga, sa)
    process(base_a + MGB, MGB, gb, sb)
    drain_scatters(0, sa)
    drain_scatters(MGB, sb)
    left = _MNGROUP * MGB
    for b in range(_MNLEFT):
        for g in range(K // 16):
            gsl = pl.ds(g * 16, 16)
            typidx_v[b, gsl] = (comb_v[left + b, gsl] & 63) + trel
            dstidx_v[b, gsl] = jnp.right_shift(comb_v[left + b, gsl], 6)
        pltpu.sync_copy(x_hbm.at[src_v.at[left + b]], gbufs[b])
        pltpu.sync_copy(relf_hbm.at[typidx_v.at[b]], rbufs[b])
        mul_chunk(gbufs[b], rbufs[b])
        pltpu.sync_copy(gbufs[b], acc.at[dstidx_v.at[b]], add=True)
    _segsum_epilogue(out0_hbm, out1_hbm, acc, cid, sid)


BM = 2000


def _tc_T_body(c0_ref, c1_ref, rp_ref, o_ref):
    o_ref[...] = jnp.dot(c0_ref[...] + c1_ref[...], rp_ref[...],
                         preferred_element_type=_F32)


def _tc_T(c0, c1, relpad):
    return pl.pallas_call(
        _tc_T_body,
        grid=(N // BM,),
        in_specs=[
            pl.BlockSpec((BM, RP), lambda i: (i, 0)),
            pl.BlockSpec((BM, RP), lambda i: (i, 0)),
            pl.BlockSpec((RP, D), lambda i: (0, 0)),
        ],
        out_specs=pl.BlockSpec((BM, D), lambda i: (i, 0)),
        out_shape=jax.ShapeDtypeStruct((N, D), _F32),
    )(c0, c1, relpad)


def _tc_step_body(p_ref, t_ref, a0_ref, a1_ref, x_ref, wm_ref, ws_ref, e_ref,
                  xo_ref, eo_ref):
    sign = p_ref[0, 0]
    w = p_ref[0, 1]
    agg = jnp.concatenate([a0_ref[...], a1_ref[...]], axis=1)
    agg = agg + sign * t_ref[...]
    h = jnp.dot(agg, wm_ref[...], preferred_element_type=_F32)
    h = h + jnp.dot(x_ref[...], ws_ref[...], preferred_element_type=_F32)
    xn = jnp.maximum(h, 0.0)
    xo_ref[...] = xn
    eo_ref[...] = e_ref[...] + w * xn


def _tc_step(params, t, a0, a1, x, wm, ws, ent_in):
    return pl.pallas_call(
        _tc_step_body,
        grid=(N // BM,),
        in_specs=[
            pl.BlockSpec((1, 2), lambda i: (0, 0)),
            pl.BlockSpec((BM, D), lambda i: (i, 0)),
            pl.BlockSpec((BM, DH), lambda i: (i, 0)),
            pl.BlockSpec((BM, DH), lambda i: (i, 0)),
            pl.BlockSpec((BM, D), lambda i: (i, 0)),
            pl.BlockSpec((D, D), lambda i: (0, 0)),
            pl.BlockSpec((D, D), lambda i: (0, 0)),
            pl.BlockSpec((BM, D), lambda i: (i, 0)),
        ],
        out_specs=[pl.BlockSpec((BM, D), lambda i: (i, 0))] * 2,
        out_shape=[jax.ShapeDtypeStruct((N, D), _F32)] * 2,
    )(params, t, a0, a1, x, wm, ws, ent_in)


def _tc_rel_body(w_ref, rel_ref, wr_ref, o_ref):
    wc = (w_ref[0, 0] * wr_ref[0] + w_ref[0, 1] * wr_ref[1]
          + w_ref[0, 2] * wr_ref[2])
    o_ref[...] = jnp.dot(rel_ref[...], wc, preferred_element_type=_F32)


def _tc_rel(w2d, rel, wr):
    return pl.pallas_call(
        _tc_rel_body,
        in_specs=[
            pl.BlockSpec((1, 3), lambda: (0, 0)),
            pl.BlockSpec((R, D), lambda: (0, 0)),
            pl.BlockSpec((3, D, D), lambda: (0, 0, 0)),
        ],
        out_specs=pl.BlockSpec((R, D), lambda: (0, 0)),
        out_shape=jax.ShapeDtypeStruct((R, D), _F32),
    )(w2d, rel, wr)


def _split(x):
    return jnp.concatenate([x[:, :DH], x[:, DH:]], axis=0)


def kernel(x, edge_index, edge_type, rel_embeds, weights, W_msg, W_self, W_rel):
    src_c = edge_index[0].reshape(NW, NCHUNKC, KC)
    dst_c = edge_index[1].reshape(NW, NCHUNKC, KC)
    typ_c = edge_type.reshape(NW, NCHUNKC, KC)
    src = edge_index[0].reshape(NS, NCHUNK, K)
    dst = edge_index[1].reshape(NS, NCHUNK, K)
    comb = dst * RP + edge_type.reshape(NS, NCHUNK, K)
    relf = jnp.repeat(_split(rel_embeds).reshape(NC, R, DH), NS, axis=0)
    relf = relf.reshape(NW * R, DH)
    relpad = jnp.zeros((RP, D), _F32).at[:R].set(rel_embeds)

    c0, c1 = _sc_count(dst_c, typ_c)
    t_mat = _tc_T(c0.reshape(N, RP), c1.reshape(N, RP), relpad)

    signs = jnp.array([-1.0, 0.0, 1.0], _F32)
    params = jnp.stack([signs, weights.astype(_F32)], axis=1)

    ent = jnp.zeros((N, D), _F32)
    a0, a1 = _sc_segsum(_split(x), src, dst)
    x1, ent = _tc_step(params[0].reshape(1, 2), t_mat, a0, a1, x,
                       W_msg[0], W_self[0], ent)
    a0, a1 = _sc_segsum_mul(_split(x1), src, comb, relf)
    x2, ent = _tc_step(params[1].reshape(1, 2), t_mat, a0, a1, x1,
                       W_msg[1], W_self[1], ent)
    a0, a1 = _sc_segsum(_split(x2), src, dst)
    _, ent = _tc_step(params[2].reshape(1, 2), t_mat, a0, a1, x2,
                      W_msg[2], W_self[2], ent)

    rel_out = _tc_rel(weights.astype(_F32).reshape(1, 3), rel_embeds, W_rel)
    return (ent, rel_out)


# speedup vs baseline: 1.3431x; 1.0069x over previous
import functools

import jax
import jax.numpy as jnp
from jax import lax
from jax.experimental import pallas as pl
from jax.experimental.pallas import tpu as pltpu
from jax.experimental.pallas import tpu_sc as plsc

N = 10000
E = 320000
D = 128
DH = D // 2
R = 50
RP = 64
NC = 2
NS = 16
NW = NC * NS
EPW = E // NW
KC = 80
NCHUNKC = EPW // KC
EPT = E // NS
K = 80
NCHUNK = EPT // K
RPT = 624
CPT = (N * RP) // NS
_TAIL = N - NS * RPT
_F32 = jnp.float32

_MESH = plsc.VectorSubcoreMesh(core_axis_name="c", subcore_axis_name="s")
_SC_PARAMS = pltpu.CompilerParams(use_tc_tiling_on_sc=False)


@functools.partial(
    pl.kernel,
    out_type=[jax.ShapeDtypeStruct((N * RP,), _F32)] * NC,
    mesh=_MESH,
    compiler_params=_SC_PARAMS,
    scratch_types=[
        pltpu.VMEM((NCHUNKC, KC), jnp.int32),
        pltpu.VMEM((NCHUNKC, KC), jnp.int32),
        pltpu.VMEM((KC,), _F32),
        pltpu.VMEM((8000,), _F32),
        pltpu.VMEM_SHARED((N * RP,), _F32),
    ],
)
def _sc_count(dst_hbm, typ_hbm, out0_hbm, out1_hbm, dst_v, fidx_v, ones_v,
              zbuf, acc):
    cid = lax.axis_index("c")
    sid = lax.axis_index("s")
    wid = cid * NS + sid
    pltpu.sync_copy(dst_hbm.at[wid], dst_v)
    pltpu.sync_copy(typ_hbm.at[wid], fidx_v)

    def fuse(j, _):
        for s in range(KC // 16):
            sl = pl.ds(s * 16, 16)
            fidx_v[j, sl] = dst_v[j, sl] * RP + fidx_v[j, sl]
        return 0

    lax.fori_loop(0, NCHUNKC, fuse, 0)

    one16 = jnp.ones((16,), _F32)
    for s in range(KC // 16):
        ones_v[pl.ds(s * 16, 16)] = one16

    zero16 = jnp.zeros((16,), _F32)

    def zb(i, _):
        zbuf[pl.ds(i * 16, 16)] = zero16
        return 0

    lax.fori_loop(0, 8000 // 16, zb, 0)
    for b in range(CPT // 8000):
        pltpu.sync_copy(zbuf, acc.at[pl.ds(sid * CPT + b * 8000, 8000)])
    plsc.subcore_barrier()

    def ch(j, _):
        pltpu.sync_copy(ones_v, acc.at[fidx_v.at[j]], add=True)
        return 0

    lax.fori_loop(0, NCHUNKC, ch, 0)
    plsc.subcore_barrier()

    @pl.when(jnp.logical_and(sid == 0, cid == 0))
    def _():
        pltpu.sync_copy(acc, out0_hbm)

    @pl.when(jnp.logical_and(sid == 0, cid == 1))
    def _():
        pltpu.sync_copy(acc, out1_hbm)


_GB = 4


def _segsum_scratch():
    return [
        pltpu.VMEM((NCHUNK, K), jnp.int32),
        pltpu.VMEM((NCHUNK, K), jnp.int32),
    ] + [pltpu.VMEM((K, DH), _F32)] * (2 * _GB) + [
        pltpu.VMEM((16, DH), _F32),
        pltpu.VMEM_SHARED((N, DH), _F32),
        pltpu.SemaphoreType.DMA,
        pltpu.SemaphoreType.DMA,
        pltpu.SemaphoreType.DMA,
        pltpu.SemaphoreType.DMA,
    ]


def _segsum_prologue(src_hbm, dst_hbm, src_v, dst_v, zbuf, acc):
    cid = lax.axis_index("c")
    sid = lax.axis_index("s")
    pltpu.sync_copy(src_hbm.at[sid], src_v)
    pltpu.sync_copy(dst_hbm.at[sid], dst_v)

    bias = cid * N

    def add_bias(j, _):
        for s in range(K // 16):
            sl = pl.ds(s * 16, 16)
            src_v[j, sl] = src_v[j, sl] + bias
        return 0

    lax.fori_loop(0, NCHUNK, add_bias, 0)

    zero16 = jnp.zeros((16,), _F32)

    def zrow(r, _):
        for s in range(DH // 16):
            zbuf[r, pl.ds(s * 16, 16)] = zero16
        return 0

    lax.fori_loop(0, 16, zrow, 0)
    for b in range(RPT // 16):
        pltpu.sync_copy(zbuf, acc.at[pl.ds(sid * RPT + b * 16, 16)])

    @pl.when(sid == NS - 1)
    def _():
        pltpu.sync_copy(zbuf.at[pl.ds(0, _TAIL)], acc.at[pl.ds(NS * RPT, _TAIL)])

    plsc.subcore_barrier()
    return cid, sid


def _segsum_epilogue(out0_hbm, out1_hbm, acc, cid, sid):
    plsc.subcore_barrier()
    for c, out in ((0, out0_hbm), (1, out1_hbm)):
        @pl.when(cid == c)
        def _(out=out):
            pltpu.sync_copy(acc.at[pl.ds(sid * RPT, RPT)],
                            out.at[pl.ds(sid * RPT, RPT)])

            @pl.when(sid == NS - 1)
            def _():
                pltpu.sync_copy(acc.at[pl.ds(NS * RPT, _TAIL)],
                                out.at[pl.ds(NS * RPT, _TAIL)])


_NGROUP = NCHUNK // _GB
_NGITER = (_NGROUP - 2) // 2
_NLEFT = NCHUNK - _NGROUP * _GB


def _segsum_pipeline(x_hbm, src_v, bufs, acc, ga, gb, sa, sb,
                     work_chunk, dst_idx):
    A, B = bufs[:_GB], bufs[_GB:]

    def fire_gathers(base, grp, sem):
        for b in range(_GB):
            pltpu.async_copy(x_hbm.at[src_v.at[base + b]], grp[b], sem)

    def drain_gathers(base, grp, sem):
        for b in range(_GB):
            pltpu.make_async_copy(x_hbm.at[src_v.at[base + b]], grp[b],
                                  sem).wait()

    def fire_scatters(base, grp, so, sem):
        for b in range(_GB):
            pltpu.async_copy(grp[b], acc.at[dst_idx(base + b, so + b)], sem,
                             add=True)

    def drain_scatters(base, grp, so, sem):
        for b in range(_GB):
            pltpu.make_async_copy(grp[b], acc.at[dst_idx(base + b, so + b)],
                                  sem).wait()

    def process(base, grp, so, gsem, ssem):
        drain_gathers(base, grp, gsem)
        for b in range(_GB):
            work_chunk(grp[b], base + b, so + b)
        fire_scatters(base, grp, so, ssem)

    fire_gathers(0, A, ga)
    fire_gathers(_GB, B, gb)

    def it(i, _):
        g0 = i * 2 * _GB
        g1 = g0 + _GB
        process(g0, A, 0, ga, sa)
        process(g1, B, _GB, gb, sb)
        drain_scatters(g0, A, 0, sa)
        fire_gathers(g0 + 2 * _GB, A, ga)
        drain_scatters(g1, B, _GB, sb)
        fire_gathers(g1 + 2 * _GB, B, gb)
        return 0

    lax.fori_loop(0, _NGITER, it, 0)
    base_a = _NGITER * 2 * _GB
    base_b = base_a + _GB
    process(base_a, A, 0, ga, sa)
    process(base_b, B, _GB, gb, sb)
    drain_scatters(base_a, A, 0, sa)
    left = _NGROUP * _GB
    for b in range(_NLEFT):
        pltpu.async_copy(x_hbm.at[src_v.at[left + b]], A[b], ga)
    drain_scatters(base_b, B, _GB, sb)
    for b in range(_NLEFT):
        pltpu.make_async_copy(x_hbm.at[src_v.at[left + b]], A[b], ga).wait()
        work_chunk(A[b], left + b, b)
        pltpu.async_copy(A[b], acc.at[dst_idx(left + b, b)], sa, add=True)
    for b in range(_NLEFT):
        pltpu.make_async_copy(A[b], acc.at[dst_idx(left + b, b)], sa).wait()


@functools.partial(
    pl.kernel,
    out_type=[jax.ShapeDtypeStruct((N, DH), _F32)] * NC,
    mesh=_MESH,
    compiler_params=_SC_PARAMS,
    scratch_types=_segsum_scratch(),
)
def _sc_segsum(x_hbm, src_hbm, dst_hbm, out0_hbm, out1_hbm,
               src_v, dst_v, b0, b1, b2, b3, b4, b5, b6, b7,
               zbuf, acc, ga, gb, sa, sb):
    cid, sid = _segsum_prologue(src_hbm, dst_hbm, src_v, dst_v, zbuf, acc)
    _segsum_pipeline(x_hbm, src_v, (b0, b1, b2, b3, b4, b5, b6, b7),
                     acc, ga, gb, sa, sb,
                     lambda gbuf, j, slot: None,
                     lambda j, slot: dst_v.at[j])
    _segsum_epilogue(out0_hbm, out1_hbm, acc, cid, sid)


MGB = 2
_MNGROUP = 2 * (NCHUNK // (2 * MGB))
_MNGITER = (_MNGROUP - 2) // 2
_MNLEFT = NCHUNK - _MNGROUP * MGB


@functools.partial(
    pl.kernel,
    out_type=[jax.ShapeDtypeStruct((N, DH), _F32)] * NC,
    mesh=_MESH,
    compiler_params=_SC_PARAMS,
    scratch_types=[
        pltpu.VMEM((NCHUNK, K), jnp.int32),
        pltpu.VMEM((NCHUNK, K), jnp.int32),
    ] + [pltpu.VMEM((K, DH), _F32)] * (2 * MGB)
      + [pltpu.VMEM((K, DH), _F32)] * (2 * MGB) + [
        pltpu.VMEM((16, DH), _F32),
        pltpu.VMEM_SHARED((N, DH), _F32),
        pltpu.SemaphoreType.DMA,
        pltpu.SemaphoreType.DMA,
        pltpu.SemaphoreType.DMA,
        pltpu.SemaphoreType.DMA,
        pltpu.VMEM((2 * MGB, K), jnp.int32),
        pltpu.VMEM((2 * MGB, K), jnp.int32),
    ],
)
def _sc_segsum_mul(x_hbm, src_hbm, comb_hbm, relf_hbm, out0_hbm, out1_hbm,
                   src_v, comb_v, g0b, g1b, g2b, g3b, r0b, r1b, r2b, r3b,
                   zbuf, acc, ga, gb, sa, sb, dstidx_v, typidx_v):
    cid = lax.axis_index("c")
    sid = lax.axis_index("s")
    pltpu.sync_copy(src_hbm.at[sid], src_v)
    pltpu.sync_copy(comb_hbm.at[sid], comb_v)

    bias = cid * N

    def add_bias(j, _):
        for s in range(K // 16):
            sl = pl.ds(s * 16, 16)
            src_v[j, sl] = src_v[j, sl] + bias
        return 0

    lax.fori_loop(0, NCHUNK, add_bias, 0)

    zero16 = jnp.zeros((16,), _F32)

    def zrow(r, _):
        for s in range(DH // 16):
            zbuf[r, pl.ds(s * 16, 16)] = zero16
        return 0

    lax.fori_loop(0, 16, zrow, 0)
    for b in range(RPT // 16):
        pltpu.sync_copy(zbuf, acc.at[pl.ds(sid * RPT + b * 16, 16)])

    @pl.when(sid == NS - 1)
    def _():
        pltpu.sync_copy(zbuf.at[pl.ds(0, _TAIL)], acc.at[pl.ds(NS * RPT, _TAIL)])

    plsc.subcore_barrier()

    gbufs = (g0b, g1b, g2b, g3b)
    rbufs = (r0b, r1b, r2b, r3b)
    trel = (cid * NS + sid) * R

    def fire_group(base, so, gsem):
        for b in range(MGB):
            for g in range(K // 16):
                gsl = pl.ds(g * 16, 16)
                typidx_v[so + b, gsl] = (comb_v[base + b, gsl] & 63) + trel
        for b in range(MGB):
            pltpu.async_copy(x_hbm.at[src_v.at[base + b]], gbufs[so + b], gsem)
        for b in range(MGB):
            pltpu.async_copy(relf_hbm.at[typidx_v.at[so + b]], rbufs[so + b],
                             gsem)

    def drain_group(base, so, gsem):
        for b in range(MGB):
            pltpu.make_async_copy(x_hbm.at[src_v.at[base + b]], gbufs[so + b],
                                  gsem).wait()
        for b in range(MGB):
            pltpu.make_async_copy(relf_hbm.at[typidx_v.at[so + b]],
                                  rbufs[so + b], gsem).wait()

    def mul_chunk(gbuf, rbuf):
        @plsc.parallel_loop(0, K, unroll=4)
        def _(rr):
            xs = [gbuf[rr, pl.ds(s * 16, 16)] for s in range(DH // 16)]
            rl = [rbuf[rr, pl.ds(s * 16, 16)] for s in range(DH // 16)]
            for s in range(DH // 16):
                gbuf[rr, pl.ds(s * 16, 16)] = xs[s] * rl[s]

    def process(base, so, gsem, ssem):
        drain_group(base, so, gsem)
        for b in range(MGB):
            for g in range(K // 16):
                gsl = pl.ds(g * 16, 16)
                dstidx_v[so + b, gsl] = jnp.right_shift(comb_v[base + b, gsl], 6)
            mul_chunk(gbufs[so + b], rbufs[so + b])
        for b in range(MGB):
            pltpu.async_copy(gbufs[so + b], acc.at[dstidx_v.at[so + b]], ssem,
                             add=True)

    def drain_scatters(so, ssem):
        for b in range(MGB):
            pltpu.make_async_copy(gbufs[so + b], acc.at[dstidx_v.at[so + b]],
                                  ssem).wait()

    fire_group(0, 0, ga)
    fire_group(MGB, MGB, gb)

    def it(i, _):
        g0 = i * 2 * MGB
        g1 = g0 + MGB
        process(g0, 0, ga, sa)
        process(g1, MGB, gb, sb)
        drain_scatters(0, sa)
        fire_group(g0 + 2 * MGB, 0, ga)
        drain_scatters(MGB, sb)
        fire_group(g1 + 2 * MGB, MGB, gb)
        return 0

    lax.fori_loop(0, _MNGITER, it, 0)
    base_a = _MNGITER * 2 * MGB
    process(base_a, 0, ga, sa)
    process(base_a + MGB, MGB, gb, sb)
    drain_scatters(0, sa)
    drain_scatters(MGB, sb)
    left = _MNGROUP * MGB
    for b in range(_MNLEFT):
        for g in range(K // 16):
            gsl = pl.ds(g * 16, 16)
            typidx_v[b, gsl] = (comb_v[left + b, gsl] & 63) + trel
            dstidx_v[b, gsl] = jnp.right_shift(comb_v[left + b, gsl], 6)
        pltpu.sync_copy(x_hbm.at[src_v.at[left + b]], gbufs[b])
        pltpu.sync_copy(relf_hbm.at[typidx_v.at[b]], rbufs[b])
        mul_chunk(gbufs[b], rbufs[b])
        pltpu.sync_copy(gbufs[b], acc.at[dstidx_v.at[b]], add=True)
    _segsum_epilogue(out0_hbm, out1_hbm, acc, cid, sid)


BM = 2000


def _tc_step_body(p_ref, c0_ref, c1_ref, rp_ref, a0_ref, a1_ref, x_ref,
                  wm_ref, ws_ref, e_ref, xo_ref, eo_ref):
    sign = p_ref[0, 0]
    w = p_ref[0, 1]
    t_blk = jnp.dot(c0_ref[...] + c1_ref[...], rp_ref[...],
                    preferred_element_type=_F32)
    agg = jnp.concatenate([a0_ref[...], a1_ref[...]], axis=1) + sign * t_blk
    h = jnp.dot(agg, wm_ref[...], preferred_element_type=_F32)
    h = h + jnp.dot(x_ref[...], ws_ref[...], preferred_element_type=_F32)
    xn = jnp.maximum(h, 0.0)
    xo_ref[...] = xn
    eo_ref[...] = e_ref[...] + w * xn


def _tc_step(params, c0, c1, relpad, a0, a1, x, wm, ws, ent_in):
    return pl.pallas_call(
        _tc_step_body,
        grid=(N // BM,),
        in_specs=[
            pl.BlockSpec((1, 2), lambda i: (0, 0)),
            pl.BlockSpec((BM, RP), lambda i: (i, 0)),
            pl.BlockSpec((BM, RP), lambda i: (i, 0)),
            pl.BlockSpec((RP, D), lambda i: (0, 0)),
            pl.BlockSpec((BM, DH), lambda i: (i, 0)),
            pl.BlockSpec((BM, DH), lambda i: (i, 0)),
            pl.BlockSpec((BM, D), lambda i: (i, 0)),
            pl.BlockSpec((D, D), lambda i: (0, 0)),
            pl.BlockSpec((D, D), lambda i: (0, 0)),
            pl.BlockSpec((BM, D), lambda i: (i, 0)),
        ],
        out_specs=[pl.BlockSpec((BM, D), lambda i: (i, 0))] * 2,
        out_shape=[jax.ShapeDtypeStruct((N, D), _F32)] * 2,
    )(params, c0, c1, relpad, a0, a1, x, wm, ws, ent_in)


def _tc_step_last_body(p_ref, c0_ref, c1_ref, rp_ref, a0_ref, a1_ref, x_ref,
                       wm_ref, ws_ref, e_ref, w3_ref, rel_ref, wr_ref,
                       eo_ref, ro_ref):
    sign = p_ref[0, 0]
    w = p_ref[0, 1]
    t_blk = jnp.dot(c0_ref[...] + c1_ref[...], rp_ref[...],
                    preferred_element_type=_F32)
    agg = jnp.concatenate([a0_ref[...], a1_ref[...]], axis=1) + sign * t_blk
    h = jnp.dot(agg, wm_ref[...], preferred_element_type=_F32)
    h = h + jnp.dot(x_ref[...], ws_ref[...], preferred_element_type=_F32)
    xn = jnp.maximum(h, 0.0)
    eo_ref[...] = e_ref[...] + w * xn

    @pl.when(pl.program_id(0) == 0)
    def _():
        wc = (w3_ref[0, 0] * wr_ref[0] + w3_ref[0, 1] * wr_ref[1]
              + w3_ref[0, 2] * wr_ref[2])
        ro_ref[...] = jnp.dot(rel_ref[...], wc, preferred_element_type=_F32)


def _tc_step_last(params, c0, c1, relpad, a0, a1, x, wm, ws, ent_in,
                  w3, rel, wr):
    return pl.pallas_call(
        _tc_step_last_body,
        grid=(N // BM,),
        in_specs=[
            pl.BlockSpec((1, 2), lambda i: (0, 0)),
            pl.BlockSpec((BM, RP), lambda i: (i, 0)),
            pl.BlockSpec((BM, RP), lambda i: (i, 0)),
            pl.BlockSpec((RP, D), lambda i: (0, 0)),
            pl.BlockSpec((BM, DH), lambda i: (i, 0)),
            pl.BlockSpec((BM, DH), lambda i: (i, 0)),
            pl.BlockSpec((BM, D), lambda i: (i, 0)),
            pl.BlockSpec((D, D), lambda i: (0, 0)),
            pl.BlockSpec((D, D), lambda i: (0, 0)),
            pl.BlockSpec((BM, D), lambda i: (i, 0)),
            pl.BlockSpec((1, 3), lambda i: (0, 0)),
            pl.BlockSpec((R, D), lambda i: (0, 0)),
            pl.BlockSpec((3, D, D), lambda i: (0, 0, 0)),
        ],
        out_specs=[pl.BlockSpec((BM, D), lambda i: (i, 0)),
                   pl.BlockSpec((R, D), lambda i: (0, 0))],
        out_shape=[jax.ShapeDtypeStruct((N, D), _F32),
                   jax.ShapeDtypeStruct((R, D), _F32)],
    )(params, c0, c1, relpad, a0, a1, x, wm, ws, ent_in, w3, rel, wr)


def _split(x):
    return jnp.concatenate([x[:, :DH], x[:, DH:]], axis=0)


def kernel(x, edge_index, edge_type, rel_embeds, weights, W_msg, W_self, W_rel):
    src_c = edge_index[0].reshape(NW, NCHUNKC, KC)
    dst_c = edge_index[1].reshape(NW, NCHUNKC, KC)
    typ_c = edge_type.reshape(NW, NCHUNKC, KC)
    src = edge_index[0].reshape(NS, NCHUNK, K)
    dst = edge_index[1].reshape(NS, NCHUNK, K)
    comb = dst * RP + edge_type.reshape(NS, NCHUNK, K)
    relf = jnp.repeat(_split(rel_embeds).reshape(NC, R, DH), NS, axis=0)
    relf = relf.reshape(NW * R, DH)
    relpad = jnp.zeros((RP, D), _F32).at[:R].set(rel_embeds)

    c0, c1 = _sc_count(dst_c, typ_c)
    c0 = c0.reshape(N, RP)
    c1 = c1.reshape(N, RP)

    signs = jnp.array([-1.0, 0.0, 1.0], _F32)
    params = jnp.stack([signs, weights.astype(_F32)], axis=1)
    w3 = weights.astype(_F32).reshape(1, 3)

    ent = jnp.zeros((N, D), _F32)
    a0, a1 = _sc_segsum(_split(x), src, dst)
    x1, ent = _tc_step(params[0].reshape(1, 2), c0, c1, relpad, a0, a1, x,
                       W_msg[0], W_self[0], ent)
    a0, a1 = _sc_segsum_mul(_split(x1), src, comb, relf)
    x2, ent = _tc_step(params[1].reshape(1, 2), c0, c1, relpad, a0, a1, x1,
                       W_msg[1], W_self[1], ent)
    a0, a1 = _sc_segsum(_split(x2), src, dst)
    ent, rel_out = _tc_step_last(params[2].reshape(1, 2), c0, c1, relpad,
                                 a0, a1, x2, W_msg[2], W_self[2], ent,
                                 w3, rel_embeds, W_rel)
    return (ent, rel_out)
